# baseline (speedup 1.0000x reference)
import math

import jax
import jax.numpy as jnp
import numpy as np
from jax import lax
from jax.experimental import pallas as pl
from jax.experimental.pallas import tpu as pltpu
from jax.experimental.pallas import tpu_sc as plsc

MAXLEN = 512
NUM_CORES = 2
NUM_SUBCORES = 16
NW = NUM_CORES * NUM_SUBCORES
LANES = 16


def _make_pe_np(emb: int) -> np.ndarray:
    pe = np.zeros((MAXLEN, emb), dtype=np.float32)
    position = np.arange(0, MAXLEN, dtype=np.float32)[:, None]
    div_term = np.exp(
        np.arange(0, emb, 2, dtype=np.float32) * -(math.log(10000.0) / emb)
    )
    pe[:, 0::2] = np.sin(position * div_term)
    pe[:, 1::2] = np.cos(position * div_term)
    return pe


def kernel(input, weight):
    B, L = input.shape
    V, D = weight.shape
    factor = math.sqrt(D)
    bpw = B // NW
    nj = D // LANES

    pe = jnp.asarray(_make_pe_np(D)[:L])

    mesh = plsc.VectorSubcoreMesh(
        core_axis_name="c",
        subcore_axis_name="s",
        num_cores=NUM_CORES,
        num_subcores=NUM_SUBCORES,
    )

    @jax.jit
    def run(idx_t, w, pe_arr):
        @pl.kernel(
            out_type=jax.ShapeDtypeStruct((L, D, B), jnp.float32),
            mesh=mesh,
            compiler_params=pltpu.CompilerParams(
                use_tc_tiling_on_sc=False, needs_layout_passes=False
            ),
            scratch_types=[
                pltpu.VMEM((L, bpw), jnp.int32),
                pltpu.VMEM((L, D), jnp.float32),
                pltpu.VMEM((2, bpw, D), jnp.float32),
                pltpu.VMEM((2, D, bpw), jnp.float32),
                pltpu.SemaphoreType.DMA,
                pltpu.SemaphoreType.DMA,
                pltpu.SemaphoreType.DMA,
                pltpu.SemaphoreType.DMA,
            ],
        )
        def body(idx_hbm, w_hbm, pe_hbm, out_hbm, idx_v, pe_v, gb, sl,
                 gsem0, gsem1, ssem0, ssem1):
            wid = lax.axis_index("s") * NUM_CORES + lax.axis_index("c")
            b0 = wid * bpw
            pltpu.sync_copy(idx_hbm.at[:, pl.ds(b0, bpw)], idx_v)
            pltpu.sync_copy(pe_hbm, pe_v)

            gsems = (gsem0, gsem1)
            ssems = (ssem0, ssem1)
            e_iotas = [
                lax.iota(jnp.int32, LANES) + j * LANES for j in range(nj)
            ]

            def gather_start(l, par):
                pltpu.make_async_copy(
                    w_hbm.at[idx_v.at[l]], gb.at[par], gsems[par]
                ).start()

            def gather_wait(l, par):
                pltpu.make_async_copy(
                    w_hbm.at[idx_v.at[l]], gb.at[par], gsems[par]
                ).wait()

            def scatter_start(l, par):
                pltpu.make_async_copy(
                    sl.at[par], out_hbm.at[l, :, pl.ds(b0, bpw)], ssems[par]
                ).start()

            def scatter_wait(l, par):
                pltpu.make_async_copy(
                    sl.at[par], out_hbm.at[l, :, pl.ds(b0, bpw)], ssems[par]
                ).wait()

            gather_start(0, 0)

            def step(l, par):
                @pl.when(l + 1 < L)
                def _():
                    gather_start(l + 1, 1 - par)

                gather_wait(l, par)

                @pl.when(l >= 2)
                def _():
                    scatter_wait(l - 2, par)

                pevs = [pe_v[l, pl.ds(j * LANES, LANES)] for j in range(nj)]

                def bbody(b, _):
                    bs = jnp.full((LANES,), b, jnp.int32)
                    for j in range(nj):
                        v = gb[par, b, pl.ds(j * LANES, LANES)] * factor
                        v = v + pevs[j]
                        plsc.store_scatter(sl.at[par], [e_iotas[j], bs], v)
                    return 0

                lax.fori_loop(0, bpw, bbody, 0)
                scatter_start(l, par)

            def loop2(l2, _):
                step(2 * l2, 0)
                step(2 * l2 + 1, 1)
                return 0

            lax.fori_loop(0, L // 2, loop2, 0)
            scatter_wait(L - 2, 0)
            scatter_wait(L - 1, 1)

        return body(idx_t, w, pe_arr)

    out = run(input.T, weight, pe)
    return out.transpose(2, 0, 1)


# speedup vs baseline: 1.5271x; 1.5271x over previous
import math

import jax
import jax.numpy as jnp
import numpy as np
from jax import lax
from jax.experimental import pallas as pl
from jax.experimental.pallas import tpu as pltpu
from jax.experimental.pallas import tpu_sc as plsc

MAXLEN = 512
NUM_CORES = 2
NUM_SUBCORES = 16
NW = NUM_CORES * NUM_SUBCORES
LANES = 16


def _make_pe_np(emb: int) -> np.ndarray:
    pe = np.zeros((MAXLEN, emb), dtype=np.float32)
    position = np.arange(0, MAXLEN, dtype=np.float32)[:, None]
    div_term = np.exp(
        np.arange(0, emb, 2, dtype=np.float32) * -(math.log(10000.0) / emb)
    )
    pe[:, 0::2] = np.sin(position * div_term)
    pe[:, 1::2] = np.cos(position * div_term)
    return pe


def kernel(input, weight):
    B, L = input.shape
    V, D = weight.shape
    factor = math.sqrt(D)
    bpw = B // NW
    rpw2 = 2 * D // 128
    ng = bpw // LANES
    ne = D

    pe = jnp.asarray(_make_pe_np(D)[:L])

    mesh = plsc.VectorSubcoreMesh(
        core_axis_name="c",
        subcore_axis_name="s",
        num_cores=NUM_CORES,
        num_subcores=NUM_SUBCORES,
    )

    @jax.jit
    def run(idx_t, w2, pe_arr):
        @pl.kernel(
            out_type=jax.ShapeDtypeStruct((L, D // 8, B // bpw, 8, bpw),
                                          jnp.float32),
            mesh=mesh,
            compiler_params=pltpu.CompilerParams(
                use_tc_tiling_on_sc=False, needs_layout_passes=False
            ),
            scratch_types=[
                pltpu.VMEM((L, bpw), jnp.int32),
                pltpu.VMEM((L, D), jnp.float32),
                pltpu.VMEM((2, bpw), jnp.int32),
                pltpu.VMEM((2, bpw, 128), jnp.float32),
                pltpu.VMEM((2, D // 8, 8, bpw), jnp.float32),
                pltpu.SemaphoreType.DMA,
                pltpu.SemaphoreType.DMA,
                pltpu.SemaphoreType.DMA,
                pltpu.SemaphoreType.DMA,
            ],
        )
        def body(idx_hbm, w_hbm, pe_hbm, out_hbm, idx_v, pe_v, rows_v, gb, sl,
                 gsem0, gsem1, ssem0, ssem1):
            wid = lax.axis_index("s") * NUM_CORES + lax.axis_index("c")
            b0 = wid * bpw
            pltpu.sync_copy(idx_hbm.at[:, pl.ds(b0, bpw)], idx_v)
            pltpu.sync_copy(pe_hbm, pe_v)

            gsems = (gsem0, gsem1)
            ssems = (ssem0, ssem1)
            biotas = [
                lax.iota(jnp.int32, LANES) + k * LANES for k in range(ng)
            ]

            def rows_compute(l, par):
                for k in range(ng):
                    cs = pl.ds(k * LANES, LANES)
                    rows_v[par, cs] = lax.shift_right_logical(
                        idx_v[l, cs], 1
                    )

            def gather_start(l, par):
                pltpu.make_async_copy(
                    w_hbm.at[rows_v.at[par]], gb.at[par], gsems[par]
                ).start()

            def gather_wait(l, par):
                pltpu.make_async_copy(
                    w_hbm.at[rows_v.at[par]], gb.at[par], gsems[par]
                ).wait()

            def scatter_start(l, par):
                pltpu.make_async_copy(
                    sl.at[par], out_hbm.at[l, :, wid], ssems[par]
                ).start()

            def scatter_wait(l, par):
                pltpu.make_async_copy(
                    sl.at[par], out_hbm.at[l, :, wid], ssems[par]
                ).wait()

            rows_compute(0, 0)
            gather_start(0, 0)

            def step(l, par):
                @pl.when(l + 1 < L)
                def _():
                    rows_compute(l + 1, 1 - par)
                    gather_start(l + 1, 1 - par)

                gather_wait(l, par)

                @pl.when(l >= 2)
                def _():
                    scatter_wait(l - 2, par)

                h64 = [
                    lax.shift_left(
                        lax.bitwise_and(
                            idx_v[l, pl.ds(k * LANES, LANES)],
                            jnp.full((LANES,), 1, jnp.int32),
                        ),
                        jnp.full((LANES,), 6, jnp.int32),
                    )
                    for k in range(ng)
                ]
                lsplat = jnp.full((LANES,), l, jnp.int32)
                gbp = gb.at[par]
                slp = sl.at[par]

                @plsc.parallel_loop(0, ne, unroll=2)
                def _(e):
                    esplat = jnp.full((LANES,), e, jnp.int32)
                    pev = plsc.load_gather(pe_v, [lsplat, esplat])
                    ehi = lax.shift_right_logical(e, 3)
                    elo = lax.bitwise_and(e, 7)
                    for k in range(ng):
                        v = plsc.load_gather(gbp, [biotas[k], h64[k] + e])
                        v = v * factor + pev
                        slp[ehi, elo, pl.ds(k * LANES, LANES)] = v

                scatter_start(l, par)

            def loop2(l2, _):
                step(2 * l2, 0)
                step(2 * l2 + 1, 1)
                return 0

            lax.fori_loop(0, L // 2, loop2, 0)
            scatter_wait(L - 2, 0)
            scatter_wait(L - 1, 1)

        return body(idx_t, w2, pe_arr)

    out6 = run(input.T, weight.reshape(V // 2, 128), pe)
    out = out6.transpose(2, 4, 0, 1, 3).reshape(B, L, D)
    return out


# speedup vs baseline: 1.5403x; 1.0086x over previous
import math

import jax
import jax.numpy as jnp
import numpy as np
from jax import lax
from jax.experimental import pallas as pl
from jax.experimental.pallas import tpu as pltpu
from jax.experimental.pallas import tpu_sc as plsc

MAXLEN = 512
NUM_CORES = 2
NUM_SUBCORES = 16
NW = NUM_CORES * NUM_SUBCORES
LANES = 16


def _make_pe_np(emb: int) -> np.ndarray:
    pe = np.zeros((MAXLEN, emb), dtype=np.float32)
    position = np.arange(0, MAXLEN, dtype=np.float32)[:, None]
    div_term = np.exp(
        np.arange(0, emb, 2, dtype=np.float32) * -(math.log(10000.0) / emb)
    )
    pe[:, 0::2] = np.sin(position * div_term)
    pe[:, 1::2] = np.cos(position * div_term)
    return pe


def kernel(input, weight):
    B, L = input.shape
    V, D = weight.shape
    factor = math.sqrt(D)
    bpw = B // NW
    ng = bpw // LANES

    pe = jnp.asarray(_make_pe_np(D)[:L])

    mesh = plsc.VectorSubcoreMesh(
        core_axis_name="c",
        subcore_axis_name="s",
        num_cores=NUM_CORES,
        num_subcores=NUM_SUBCORES,
    )

    @jax.jit
    def run(idx_t, w2, pe_arr):
        @pl.kernel(
            out_type=jax.ShapeDtypeStruct((L, D // 8, B // bpw, 8, bpw),
                                          jnp.float32),
            mesh=mesh,
            compiler_params=pltpu.CompilerParams(
                use_tc_tiling_on_sc=False, needs_layout_passes=False
            ),
            scratch_types=[
                pltpu.VMEM((L, bpw), jnp.int32),
                pltpu.VMEM((L * D,), jnp.float32),
                pltpu.VMEM((2, bpw), jnp.int32),
                pltpu.VMEM((2, bpw, 128), jnp.float32),
                pltpu.VMEM((2, D // 8, 8, bpw), jnp.float32),
                pltpu.SemaphoreType.DMA,
                pltpu.SemaphoreType.DMA,
                pltpu.SemaphoreType.DMA,
                pltpu.SemaphoreType.DMA,
            ],
        )
        def body(idx_hbm, w_hbm, pe_hbm, out_hbm, idx_v, pe_v, rows_v, gb, sl,
                 gsem0, gsem1, ssem0, ssem1):
            wid = lax.axis_index("s") * NUM_CORES + lax.axis_index("c")
            b0 = wid * bpw
            pltpu.sync_copy(idx_hbm.at[:, pl.ds(b0, bpw)], idx_v)
            pltpu.sync_copy(pe_hbm, pe_v)

            gsems = (gsem0, gsem1)
            ssems = (ssem0, ssem1)
            biotas = [
                lax.iota(jnp.int32, LANES) + k * LANES for k in range(ng)
            ]
            one16 = jnp.full((LANES,), 1, jnp.int32)
            six16 = jnp.full((LANES,), 6, jnp.int32)

            def rows_compute(l, par):
                for k in range(ng):
                    cs = pl.ds(k * LANES, LANES)
                    rows_v[par, cs] = lax.shift_right_logical(
                        idx_v[l, cs], 1
                    )

            def gather_start(l, par):
                pltpu.make_async_copy(
                    w_hbm.at[rows_v.at[par]], gb.at[par], gsems[par]
                ).start()

            def gather_wait(l, par):
                pltpu.make_async_copy(
                    w_hbm.at[rows_v.at[par]], gb.at[par], gsems[par]
                ).wait()

            def scatter_start(l, par):
                pltpu.make_async_copy(
                    sl.at[par], out_hbm.at[l, :, wid], ssems[par]
                ).start()

            def scatter_wait(l, par):
                pltpu.make_async_copy(
                    sl.at[par], out_hbm.at[l, :, wid], ssems[par]
                ).wait()

            rows_compute(0, 0)
            gather_start(0, 0)

            def step(l, par):
                @pl.when(l + 1 < L)
                def _():
                    rows_compute(l + 1, 1 - par)
                    gather_start(l + 1, 1 - par)

                gather_wait(l, par)

                @pl.when(l >= 2)
                def _():
                    scatter_wait(l - 2, par)

                h64 = [
                    lax.shift_left(
                        lax.bitwise_and(
                            idx_v[l, pl.ds(k * LANES, LANES)], one16
                        ),
                        six16,
                    )
                    for k in range(ng)
                ]
                l64 = l * D
                gbp = gb.at[par]
                slp = sl.at[par]

                @plsc.parallel_loop(0, D, unroll=4)
                def _(e):
                    pev = plsc.load_gather(
                        pe_v, [jnp.full((LANES,), l64 + e, jnp.int32)]
                    )
                    ehi = lax.shift_right_logical(e, 3)
                    elo = lax.bitwise_and(e, 7)
                    for k in range(ng):
                        v = plsc.load_gather(gbp, [biotas[k], h64[k] + e])
                        v = v * factor + pev
                        slp[ehi, elo, pl.ds(k * LANES, LANES)] = v

                scatter_start(l, par)

            def loop2(l2, _):
                step(2 * l2, 0)
                step(2 * l2 + 1, 1)
                return 0

            lax.fori_loop(0, L // 2, loop2, 0)
            scatter_wait(L - 2, 0)
            scatter_wait(L - 1, 1)

        return body(idx_t, w2, pe_arr)

    out6 = run(input.T, weight.reshape(V // 2, 128), pe.reshape(-1))
    out = out6.transpose(2, 4, 0, 1, 3).reshape(B, L, D)
    return out
